# baseline (speedup 1.0000x reference)
import functools

import jax
import jax.numpy as jnp
from jax import lax
from jax.experimental import pallas as pl
from jax.experimental.pallas import tpu as pltpu
from jax.experimental.pallas import tpu_sc as plsc

_EMB = 128
_NPOS = 8
_NODES = 262144
_GRAPHS = 1024
_SEG = _NPOS * _GRAPHS


_NB = 4096
_NBLK = _NODES // _NB


def _assign_body(batch_ref, protos_ref, x_ref, sb_ref):
    xb16 = x_ref[...].astype(jnp.bfloat16)
    pb = protos_ref[...]
    pb16 = pb.astype(jnp.bfloat16)
    hp = lax.dot_general(pb16, xb16, (((1,), (1,)), ((), ())),
                         preferred_element_type=jnp.float32)
    pn = jnp.sqrt(jnp.sum(pb * pb, axis=1, keepdims=True))
    s = hp / pn
    m = jnp.max(s, axis=0, keepdims=True)
    rows = lax.broadcasted_iota(jnp.int32, (_NPOS, _NB), 0)
    cand = jnp.where(s == m, rows, _NPOS)
    amin = jnp.min(cand, axis=0, keepdims=True)
    sb_ref[0] = batch_ref[0] * _NPOS + amin


def _assign(x, batch3, protos, blk0, nblk):
    sb3 = pl.pallas_call(
        _assign_body,
        grid=(nblk,),
        in_specs=[
            pl.BlockSpec((1, 1, _NB), lambda i: (i + blk0, 0, 0)),
            pl.BlockSpec((_NPOS, _EMB), lambda i: (0, 0)),
            pl.BlockSpec((_NB, _EMB), lambda i: (i + blk0, 0)),
        ],
        out_specs=pl.BlockSpec((1, 1, _NB), lambda i: (i, 0, 0)),
        out_shape=jax.ShapeDtypeStruct((nblk, 1, _NB), jnp.int32),
    )(batch3, protos, x)
    return sb3.reshape(nblk * _NB)


_CH = 128
_PER_TILE = _NODES // 32
_STEPS = _PER_TILE // _CH
_ZROWS = _SEG // 16


def _sc_pool_body(chunk_base, chunk_nodes,
                  x_hbm, sb_hbm, z_hbm, out_hbm,
                  rows0, rows1, idx0, idx1, sem0, sem1, acc_sh):
    per_tile = chunk_nodes // 32
    steps = per_tile // _CH
    cid = lax.axis_index("c")
    sid = lax.axis_index("s")
    pltpu.sync_copy(z_hbm, acc_sh.at[pl.ds(sid * _ZROWS, _ZROWS)])
    plsc.subcore_barrier()

    tile0 = cid * (chunk_nodes // 2) + sid * per_tile

    def fire(k, rows, idx, sem):
        b = pl.multiple_of(tile0 + k * _CH, _CH)
        pltpu.async_copy(x_hbm.at[pl.ds(chunk_base + b, _CH)], rows, sem)
        pltpu.async_copy(sb_hbm.at[pl.ds(b, _CH)], idx, sem)

    def drain(k, rows, idx, sem):
        b = pl.multiple_of(tile0 + k * _CH, _CH)
        pltpu.make_async_copy(x_hbm.at[pl.ds(chunk_base + b, _CH)], rows, sem).wait()
        pltpu.make_async_copy(sb_hbm.at[pl.ds(b, _CH)], idx, sem).wait()

    fire(0, rows0, idx0, sem0)

    def step(g, carry):
        bufs = ((rows0, idx0, sem0), (rows1, idx1, sem1))
        for b in (0, 1):
            k = g * 2 + b
            rows, idx, sem = bufs[b]
            nrows, nidx, nsem = bufs[1 - b]
            drain(k, rows, idx, sem)
            if b == 0:
                fire(k + 1, nrows, nidx, nsem)
            else:
                @pl.when(g < steps // 2 - 1)
                def _():
                    fire(k + 1, nrows, nidx, nsem)
            pltpu.sync_copy(rows, acc_sh.at[idx], add=True)
        return carry

    lax.fori_loop(0, steps // 2, step, 0)
    plsc.subcore_barrier()
    off = cid * _SEG + sid * _ZROWS
    pltpu.sync_copy(acc_sh.at[pl.ds(sid * _ZROWS, _ZROWS)],
                    out_hbm.at[pl.ds(off, _ZROWS)])


@functools.cache
def _sc_pool(chunk_base, chunk_nodes):
    return pl.kernel(
        functools.partial(_sc_pool_body, chunk_base, chunk_nodes),
        out_type=jax.ShapeDtypeStruct((2 * _SEG, _EMB), jnp.float32),
        mesh=plsc.VectorSubcoreMesh(core_axis_name="c", subcore_axis_name="s"),
        scratch_types=[
            pltpu.VMEM((_CH, _EMB), jnp.float32),
            pltpu.VMEM((_CH, _EMB), jnp.float32),
            pltpu.VMEM((_CH,), jnp.int32),
            pltpu.VMEM((_CH,), jnp.int32),
            pltpu.SemaphoreType.DMA,
            pltpu.SemaphoreType.DMA,
            pltpu.VMEM_SHARED((_SEG, _EMB), jnp.float32),
        ],
    )


_MB = 512


def _merge_body(*refs):
    o_ref = refs[-1]
    acc = refs[0][...]
    for r in refs[1:-1]:
        acc = acc + r[...]
    o_ref[...] = acc


def _merge(partials):
    in_specs = []
    args = []
    for p in partials:
        in_specs.append(pl.BlockSpec((_MB, _EMB), lambda i: (i, 0)))
        in_specs.append(pl.BlockSpec((_MB, _EMB), lambda i: (i + _SEG // _MB, 0)))
        args += [p, p]
    return pl.pallas_call(
        _merge_body,
        grid=(_SEG // _MB,),
        in_specs=in_specs,
        out_specs=pl.BlockSpec((_MB, _EMB), lambda i: (i, 0)),
        out_shape=jax.ShapeDtypeStruct((_SEG, _EMB), jnp.float32),
    )(*args)


_NCHUNK = 4
_CBLK = _NBLK // _NCHUNK
_CNODES = _NODES // _NCHUNK


def kernel(x, batch, protos):
    batch3 = batch.reshape(_NBLK, 1, _NB)
    zeros = jnp.zeros((_ZROWS, _EMB), jnp.float32)
    sbs = [_assign(x, batch3, protos, c * _CBLK, _CBLK)
           for c in range(_NCHUNK)]
    partials = [_sc_pool(c * _CNODES, _CNODES)(x, sbs[c], zeros)
                for c in range(_NCHUNK)]
    pooled = _merge(partials)
    return pooled.reshape(_GRAPHS, _NPOS * _EMB)


# speedup vs baseline: 1.1297x; 1.1297x over previous
import functools

import jax
import jax.numpy as jnp
from jax import lax
from jax.experimental import pallas as pl
from jax.experimental.pallas import tpu as pltpu
from jax.experimental.pallas import tpu_sc as plsc

_EMB = 128
_NPOS = 8
_NODES = 262144
_GRAPHS = 1024
_SEG = _NPOS * _GRAPHS


_NB = 4096
_NBLK = _NODES // _NB


def _assign_body(batch_ref, protos_ref, x_ref, sb_ref):
    xb16 = x_ref[...].astype(jnp.bfloat16)
    pb = protos_ref[...]
    pb16 = pb.astype(jnp.bfloat16)
    hp = lax.dot_general(pb16, xb16, (((1,), (1,)), ((), ())),
                         preferred_element_type=jnp.float32)
    pn = jnp.sqrt(jnp.sum(pb * pb, axis=1, keepdims=True))
    s = hp / pn
    m = jnp.max(s, axis=0, keepdims=True)
    rows = lax.broadcasted_iota(jnp.int32, (_NPOS, _NB), 0)
    cand = jnp.where(s == m, rows, _NPOS)
    amin = jnp.min(cand, axis=0, keepdims=True)
    sb_ref[0] = batch_ref[0] * _NPOS + amin


def _assign(x, batch3, protos, blk0, nblk):
    sb3 = pl.pallas_call(
        _assign_body,
        grid=(nblk,),
        in_specs=[
            pl.BlockSpec((1, 1, _NB), lambda i: (i + blk0, 0, 0)),
            pl.BlockSpec((_NPOS, _EMB), lambda i: (0, 0)),
            pl.BlockSpec((_NB, _EMB), lambda i: (i + blk0, 0)),
        ],
        out_specs=pl.BlockSpec((1, 1, _NB), lambda i: (i, 0, 0)),
        out_shape=jax.ShapeDtypeStruct((nblk, 1, _NB), jnp.int32),
    )(batch3, protos, x)
    return sb3.reshape(nblk * _NB)


_CH = 128
_PER_TILE = _NODES // 32
_STEPS = _PER_TILE // _CH
_ZROWS = _SEG // 16


def _sc_pool_body(chunk_base, chunk_nodes,
                  x_hbm, sb_hbm, z_hbm, out_hbm,
                  rows0, rows1, idx0, idx1, sem0, sem1, acc_sh):
    per_tile = chunk_nodes // 32
    steps = per_tile // _CH
    cid = lax.axis_index("c")
    sid = lax.axis_index("s")
    pltpu.sync_copy(z_hbm, acc_sh.at[pl.ds(sid * _ZROWS, _ZROWS)])
    plsc.subcore_barrier()

    tile0 = cid * (chunk_nodes // 2) + sid * per_tile

    def fire(k, rows, idx, sem):
        b = pl.multiple_of(tile0 + k * _CH, _CH)
        pltpu.async_copy(x_hbm.at[pl.ds(chunk_base + b, _CH)], rows, sem)
        pltpu.async_copy(sb_hbm.at[pl.ds(b, _CH)], idx, sem)

    def drain(k, rows, idx, sem):
        b = pl.multiple_of(tile0 + k * _CH, _CH)
        pltpu.make_async_copy(x_hbm.at[pl.ds(chunk_base + b, _CH)], rows, sem).wait()
        pltpu.make_async_copy(sb_hbm.at[pl.ds(b, _CH)], idx, sem).wait()

    fire(0, rows0, idx0, sem0)

    def step(g, carry):
        bufs = ((rows0, idx0, sem0), (rows1, idx1, sem1))
        for b in (0, 1):
            k = g * 2 + b
            rows, idx, sem = bufs[b]
            nrows, nidx, nsem = bufs[1 - b]
            drain(k, rows, idx, sem)
            if b == 0:
                fire(k + 1, nrows, nidx, nsem)
            else:
                @pl.when(g < steps // 2 - 1)
                def _():
                    fire(k + 1, nrows, nidx, nsem)
            pltpu.sync_copy(rows, acc_sh.at[idx], add=True)
        return carry

    lax.fori_loop(0, steps // 2, step, 0)
    plsc.subcore_barrier()
    off = cid * _SEG + sid * _ZROWS
    pltpu.sync_copy(acc_sh.at[pl.ds(sid * _ZROWS, _ZROWS)],
                    out_hbm.at[pl.ds(off, _ZROWS)])


@functools.cache
def _sc_pool(chunk_base, chunk_nodes):
    return pl.kernel(
        functools.partial(_sc_pool_body, chunk_base, chunk_nodes),
        out_type=jax.ShapeDtypeStruct((2 * _SEG, _EMB), jnp.float32),
        mesh=plsc.VectorSubcoreMesh(core_axis_name="c", subcore_axis_name="s"),
        scratch_types=[
            pltpu.VMEM((_CH, _EMB), jnp.float32),
            pltpu.VMEM((_CH, _EMB), jnp.float32),
            pltpu.VMEM((_CH,), jnp.int32),
            pltpu.VMEM((_CH,), jnp.int32),
            pltpu.SemaphoreType.DMA,
            pltpu.SemaphoreType.DMA,
            pltpu.VMEM_SHARED((_SEG, _EMB), jnp.float32),
        ],
    )


_MB = 512


def _merge_body(*refs):
    o_ref = refs[-1]
    acc = refs[0][...]
    for r in refs[1:-1]:
        acc = acc + r[...]
    o_ref[...] = acc


def _merge(partials):
    in_specs = []
    args = []
    for p in partials:
        in_specs.append(pl.BlockSpec((_MB, _EMB), lambda i: (i, 0)))
        in_specs.append(pl.BlockSpec((_MB, _EMB), lambda i: (i + _SEG // _MB, 0)))
        args += [p, p]
    return pl.pallas_call(
        _merge_body,
        grid=(_SEG // _MB,),
        in_specs=in_specs,
        out_specs=pl.BlockSpec((_MB, _EMB), lambda i: (i, 0)),
        out_shape=jax.ShapeDtypeStruct((_SEG, _EMB), jnp.float32),
    )(*args)


_NCHUNK = 2
_CBLK = _NBLK // _NCHUNK
_CNODES = _NODES // _NCHUNK


def kernel(x, batch, protos):
    batch3 = batch.reshape(_NBLK, 1, _NB)
    zeros = jnp.zeros((_ZROWS, _EMB), jnp.float32)
    sbs = [_assign(x, batch3, protos, c * _CBLK, _CBLK)
           for c in range(_NCHUNK)]
    partials = [_sc_pool(c * _CNODES, _CNODES)(x, sbs[c], zeros)
                for c in range(_NCHUNK)]
    pooled = _merge(partials)
    return pooled.reshape(_GRAPHS, _NPOS * _EMB)
